# baseline (speedup 1.0000x reference)
import jax
import jax.numpy as jnp
from jax import lax
from jax.experimental import pallas as pl
from jax.experimental.pallas import tpu as pltpu
from jax.experimental.pallas import tpu_sc as plsc

N = 10000
E = 320000
D = 128
H = 128
C = 64
G = 128

NC = 2
NS = 16
NW = NC * NS
CH = 128
K = 80
K0 = 160
K1 = 0
GK = 16
EPAD = NW * K * CH
NPAD = 10240
RPT = NPAD // NS
BR = 1000
NB = N // BR

def _mesh():
    return plsc.VectorSubcoreMesh(core_axis_name="c", subcore_axis_name="s",
                                  num_cores=NC, num_subcores=NS)


def _deg_body(dst_hbm, zeros_hbm, ones_hbm, degp_hbm, dstv, onesv, acc):
    c = lax.axis_index("c")
    s = lax.axis_index("s")
    w = s * NC + c
    pltpu.sync_copy(ones_hbm, onesv)
    pltpu.sync_copy(zeros_hbm.at[pl.ds(s * RPT, RPT)], acc.at[pl.ds(s * RPT, RPT)])
    plsc.subcore_barrier()

    def group(g, carry):
        pltpu.sync_copy(dst_hbm.at[pl.ds(w * K + g * GK, GK)], dstv)

        def step(j, carry2):
            pltpu.sync_copy(onesv, acc.at[dstv.at[j]], add=True)
            return carry2

        lax.fori_loop(0, GK, step, 0)
        return carry

    lax.fori_loop(0, K // GK, group, 0)
    plsc.subcore_barrier()
    pltpu.sync_copy(acc.at[pl.ds(s * RPT, RPT)], degp_hbm.at[c, pl.ds(s * RPT, RPT)])


def _sc_deg(dstp, zeros_h, ones_h):
    return pl.kernel(
        _deg_body,
        out_type=jax.ShapeDtypeStruct((NC, NPAD, H), jnp.float32),
        mesh=_mesh(),
        scratch_types=[
            pltpu.VMEM((GK, CH), jnp.int32),
            pltpu.VMEM((CH, H), jnp.float32),
            pltpu.VMEM_SHARED((NPAD, H), jnp.float32),
        ],
    )(dstp, zeros_h, ones_h)


def _agg_body(h_hbm, src_hbm, dst_hbm, out_hbm,
              srcv, dstv, rows0, rows1, acc, gsem0, gsem1, ssem0, ssem1):
    c = lax.axis_index("c")
    s = lax.axis_index("s")
    kc = K0 - c * (K0 - K1)
    base = c * (NS * K0) + s * kc
    @pl.when(s != NS - 1)
    def _():
        pltpu.sync_copy(h_hbm.at[pl.ds(s * RPT, RPT)], acc.at[pl.ds(s * RPT, RPT)])

    @pl.when(s == NS - 1)
    def _():
        pltpu.sync_copy(h_hbm.at[pl.ds((NS - 1) * RPT, N - (NS - 1) * RPT)],
                        acc.at[pl.ds((NS - 1) * RPT, N - (NS - 1) * RPT)])
        pltpu.sync_copy(h_hbm.at[pl.ds(0, NPAD - N)], acc.at[pl.ds(N, NPAD - N)])

    plsc.subcore_barrier()

    def group(g, carry):
        pltpu.sync_copy(src_hbm.at[pl.ds(base + g * GK, GK)], srcv)
        pltpu.sync_copy(dst_hbm.at[pl.ds(base + g * GK, GK)], dstv)

        pltpu.async_copy(h_hbm.at[srcv.at[0]], rows0, gsem0)
        pltpu.async_copy(h_hbm.at[srcv.at[1]], rows1, gsem1)

        def step(j, carry2):
            a = 2 * j
            pltpu.make_async_copy(h_hbm.at[srcv.at[a]], rows0, gsem0).wait()
            pltpu.async_copy(rows0, acc.at[dstv.at[a]], ssem0, add=True)
            pltpu.make_async_copy(h_hbm.at[srcv.at[a + 1]], rows1, gsem1).wait()
            pltpu.async_copy(rows1, acc.at[dstv.at[a + 1]], ssem1, add=True)

            @pl.when(a + 2 < GK)
            def _():
                pltpu.make_async_copy(rows0, acc.at[dstv.at[a]], ssem0).wait()
                pltpu.async_copy(h_hbm.at[srcv.at[a + 2]], rows0, gsem0)
                pltpu.make_async_copy(rows1, acc.at[dstv.at[a + 1]], ssem1).wait()
                pltpu.async_copy(h_hbm.at[srcv.at[a + 3]], rows1, gsem1)

            return carry2

        lax.fori_loop(0, GK // 2, step, 0)
        pltpu.make_async_copy(rows0, acc.at[dstv.at[GK - 2]], ssem0).wait()
        pltpu.make_async_copy(rows1, acc.at[dstv.at[GK - 1]], ssem1).wait()
        return carry

    lax.fori_loop(0, kc // GK, group, 0)
    plsc.subcore_barrier()
    pltpu.sync_copy(acc.at[pl.ds(s * RPT, RPT)], out_hbm.at[c, pl.ds(s * RPT, RPT)])


def _sc_agg(hp, srcp, dstp):
    return pl.kernel(
        _agg_body,
        out_type=jax.ShapeDtypeStruct((NC, NPAD, H), jnp.float32),
        mesh=_mesh(),
        scratch_types=[
            pltpu.VMEM((GK, CH), jnp.int32),
            pltpu.VMEM((GK, CH), jnp.int32),
            pltpu.VMEM((CH, H), jnp.float32),
            pltpu.VMEM((CH, H), jnp.float32),
            pltpu.VMEM_SHARED((NPAD, H), jnp.float32),
            pltpu.SemaphoreType.DMA,
            pltpu.SemaphoreType.DMA,
            pltpu.SemaphoreType.DMA,
            pltpu.SemaphoreType.DMA,
        ],
    )(hp, srcp, dstp)


def _tc_a_body(x_ref, w_ref, d0_ref, d1_ref, dis_ref, h_ref):
    deg = d0_ref[...] + d1_ref[...] + 1.0
    dis = lax.rsqrt(deg)
    dis_ref[...] = dis
    h_ref[...] = dis * jnp.dot(x_ref[...], w_ref[...],
                               preferred_element_type=jnp.float32)


def _tc_a(x, w1, d0, d1):
    return pl.pallas_call(
        _tc_a_body,
        grid=(NB,),
        in_specs=[
            pl.BlockSpec((BR, D), lambda i: (i, 0)),
            pl.BlockSpec((D, H), lambda i: (0, 0)),
            pl.BlockSpec((BR, 1), lambda i: (i, 0)),
            pl.BlockSpec((BR, 1), lambda i: (i, 0)),
        ],
        out_specs=[
            pl.BlockSpec((BR, 1), lambda i: (i, 0)),
            pl.BlockSpec((BR, H), lambda i: (i, 0)),
        ],
        out_shape=[
            jax.ShapeDtypeStruct((N, 1), jnp.float32),
            jax.ShapeDtypeStruct((N, H), jnp.float32),
        ],
    )(x, w1, d0, d1)


def _tc_bc_body(a0_ref, a1_ref, hp_ref, dis_ref, b_ref, w_ref, out_ref):
    dis = dis_ref[...]
    h = jnp.maximum(dis * (a0_ref[...] + a1_ref[...] - hp_ref[...]) + b_ref[...], 0.0)
    out_ref[...] = dis * jnp.dot(h, w_ref[...], preferred_element_type=jnp.float32)


def _tc_bc(a0, a1, hp, dis, b, w):
    return pl.pallas_call(
        _tc_bc_body,
        grid=(NB,),
        in_specs=[
            pl.BlockSpec((BR, H), lambda i: (i, 0)),
            pl.BlockSpec((BR, H), lambda i: (i, 0)),
            pl.BlockSpec((BR, H), lambda i: (i, 0)),
            pl.BlockSpec((BR, 1), lambda i: (i, 0)),
            pl.BlockSpec((1, H), lambda i: (0, 0)),
            pl.BlockSpec((H, H), lambda i: (0, 0)),
        ],
        out_specs=pl.BlockSpec((BR, H), lambda i: (i, 0)),
        out_shape=jax.ShapeDtypeStruct((N, H), jnp.float32),
    )(a0, a1, hp, dis, b, w)


def _tc_d_body(a0_ref, a1_ref, hp_ref, dis_ref, b_ref, bt_ref,
               l1w_ref, l1b_ref, l2w_ref, l2b_ref, out_ref, sums, cnts):
    i = pl.program_id(0)

    @pl.when(i == 0)
    def _():
        sums[...] = jnp.zeros_like(sums)
        cnts[...] = jnp.zeros_like(cnts)

    dis = dis_ref[...]
    h = jnp.maximum(dis * (a0_ref[...] + a1_ref[...] - hp_ref[...]) + b_ref[...], 0.0)
    gi = lax.broadcasted_iota(jnp.int32, (BR, G), 1)
    S = (bt_ref[...] == gi).astype(jnp.float32)
    dn = (((0,), (0,)), ((), ()))
    sums[...] += lax.dot_general(S, h, dn, preferred_element_type=jnp.float32)
    cnts[...] += lax.dot_general(S, jnp.ones_like(h), dn,
                                 preferred_element_type=jnp.float32)

    @pl.when(i == NB - 1)
    def _():
        pooled = sums[...] / jnp.maximum(cnts[...], 1.0)
        z = jnp.maximum(jnp.dot(pooled, l1w_ref[...],
                                preferred_element_type=jnp.float32) + l1b_ref[...], 0.0)
        z = jnp.dot(z, l2w_ref[...], preferred_element_type=jnp.float32) + l2b_ref[...]
        m = jnp.max(z, axis=-1, keepdims=True)
        out_ref[...] = (z - m) - jnp.log(jnp.sum(jnp.exp(z - m), axis=-1, keepdims=True))


def _tc_d(a0, a1, hp, dis, b, bt, l1w, l1b, l2w, l2b):
    return pl.pallas_call(
        _tc_d_body,
        grid=(NB,),
        in_specs=[
            pl.BlockSpec((BR, H), lambda i: (i, 0)),
            pl.BlockSpec((BR, H), lambda i: (i, 0)),
            pl.BlockSpec((BR, H), lambda i: (i, 0)),
            pl.BlockSpec((BR, 1), lambda i: (i, 0)),
            pl.BlockSpec((1, H), lambda i: (0, 0)),
            pl.BlockSpec((BR, 1), lambda i: (i, 0)),
            pl.BlockSpec((H, H), lambda i: (0, 0)),
            pl.BlockSpec((1, H), lambda i: (0, 0)),
            pl.BlockSpec((H, C), lambda i: (0, 0)),
            pl.BlockSpec((1, C), lambda i: (0, 0)),
        ],
        out_specs=pl.BlockSpec((G, C), lambda i: (0, 0)),
        out_shape=jax.ShapeDtypeStruct((G, C), jnp.float32),
        scratch_shapes=[
            pltpu.VMEM((G, H), jnp.float32),
            pltpu.VMEM((G, H), jnp.float32),
        ],
    )(a0, a1, hp, dis, b, bt, l1w, l1b, l2w, l2b)


def kernel(x, edge_index, batch, target_size, W1, b1, W2, b2, W3, b3,
           lin1_w, lin1_b, lin2_w, lin2_b):
    del target_size
    pad = EPAD - E
    srcp = jnp.concatenate(
        [edge_index[0], jnp.zeros((pad,), edge_index.dtype)]).reshape(NW * K, CH)
    dstp = jnp.concatenate(
        [edge_index[1], jnp.full((pad,), N, edge_index.dtype)]).reshape(NW * K, CH)

    zeros_h = jnp.zeros((NPAD, H), jnp.float32)
    ones_h = jnp.ones((CH, H), jnp.float32)
    degp = _sc_deg(dstp, zeros_h, ones_h)
    d0 = degp[0, :N, 0:1]
    d1 = degp[1, :N, 0:1]

    dis, h1p = _tc_a(x, W1, d0, d1)
    a1 = _sc_agg(h1p, srcp, dstp)
    h2p = _tc_bc(a1[0, :N], a1[1, :N], h1p, dis, b1.reshape(1, H), W2)
    a2 = _sc_agg(h2p, srcp, dstp)
    h3p = _tc_bc(a2[0, :N], a2[1, :N], h2p, dis, b2.reshape(1, H), W3)
    a3 = _sc_agg(h3p, srcp, dstp)
    return _tc_d(a3[0, :N], a3[1, :N], h3p, dis, b3.reshape(1, H), batch.reshape(N, 1),
                 lin1_w, lin1_b.reshape(1, H), lin2_w, lin2_b.reshape(1, C))


# speedup vs baseline: 1.5859x; 1.5859x over previous
import jax
import jax.numpy as jnp
from jax import lax
from jax.experimental import pallas as pl
from jax.experimental.pallas import tpu as pltpu
from jax.experimental.pallas import tpu_sc as plsc

N = 10000
E = 320000
D = 128
H = 128
C = 64
G = 128

NC = 2
NS = 16
NW = NC * NS
CH = 128
K = 80
K0 = 152
K1 = 8
GK = 8
EPAD = NW * K * CH
NPAD = 10240
RPT = NPAD // NS
BR = 1000
NB = N // BR

def _mesh():
    return plsc.VectorSubcoreMesh(core_axis_name="c", subcore_axis_name="s",
                                  num_cores=NC, num_subcores=NS)


def _deg_body(dst_hbm, zeros_hbm, ones_hbm, degp_hbm, dstv, onesv, acc):
    c = lax.axis_index("c")
    s = lax.axis_index("s")
    w = s * NC + c
    pltpu.sync_copy(ones_hbm, onesv)
    pltpu.sync_copy(zeros_hbm.at[pl.ds(s * RPT, RPT)], acc.at[pl.ds(s * RPT, RPT)])
    plsc.subcore_barrier()

    def group(g, carry):
        pltpu.sync_copy(dst_hbm.at[pl.ds(w * K + g * GK, GK)], dstv)

        def step(j, carry2):
            pltpu.sync_copy(onesv, acc.at[dstv.at[j]], add=True)
            return carry2

        lax.fori_loop(0, GK, step, 0)
        return carry

    lax.fori_loop(0, K // GK, group, 0)
    plsc.subcore_barrier()
    pltpu.sync_copy(acc.at[pl.ds(s * RPT, RPT)], degp_hbm.at[c, pl.ds(s * RPT, RPT)])


def _sc_deg(dstp, zeros_h, ones_h):
    return pl.kernel(
        _deg_body,
        out_type=jax.ShapeDtypeStruct((NC, NPAD, H), jnp.float32),
        mesh=_mesh(),
        scratch_types=[
            pltpu.VMEM((GK, CH), jnp.int32),
            pltpu.VMEM((CH, H), jnp.float32),
            pltpu.VMEM_SHARED((NPAD, H), jnp.float32),
        ],
    )(dstp, zeros_h, ones_h)


def _agg_body(h_hbm, src_hbm, dst_hbm, out_hbm,
              srcv, dstv, rows0, rows1, acc, gsem0, gsem1, ssem0, ssem1):
    c = lax.axis_index("c")
    s = lax.axis_index("s")
    kc = K0 - c * (K0 - K1)
    base = c * (NS * K0) + s * kc
    @pl.when(s != NS - 1)
    def _():
        pltpu.sync_copy(h_hbm.at[pl.ds(s * RPT, RPT)], acc.at[pl.ds(s * RPT, RPT)])

    @pl.when(s == NS - 1)
    def _():
        pltpu.sync_copy(h_hbm.at[pl.ds((NS - 1) * RPT, N - (NS - 1) * RPT)],
                        acc.at[pl.ds((NS - 1) * RPT, N - (NS - 1) * RPT)])
        pltpu.sync_copy(h_hbm.at[pl.ds(0, NPAD - N)], acc.at[pl.ds(N, NPAD - N)])

    plsc.subcore_barrier()

    def group(g, carry):
        pltpu.sync_copy(src_hbm.at[pl.ds(base + g * GK, GK)], srcv)
        pltpu.sync_copy(dst_hbm.at[pl.ds(base + g * GK, GK)], dstv)

        pltpu.async_copy(h_hbm.at[srcv.at[0]], rows0, gsem0)
        pltpu.async_copy(h_hbm.at[srcv.at[1]], rows1, gsem1)

        def step(j, carry2):
            a = 2 * j
            pltpu.make_async_copy(h_hbm.at[srcv.at[a]], rows0, gsem0).wait()
            pltpu.async_copy(rows0, acc.at[dstv.at[a]], ssem0, add=True)
            pltpu.make_async_copy(h_hbm.at[srcv.at[a + 1]], rows1, gsem1).wait()
            pltpu.async_copy(rows1, acc.at[dstv.at[a + 1]], ssem1, add=True)

            @pl.when(a + 2 < GK)
            def _():
                pltpu.make_async_copy(rows0, acc.at[dstv.at[a]], ssem0).wait()
                pltpu.async_copy(h_hbm.at[srcv.at[a + 2]], rows0, gsem0)
                pltpu.make_async_copy(rows1, acc.at[dstv.at[a + 1]], ssem1).wait()
                pltpu.async_copy(h_hbm.at[srcv.at[a + 3]], rows1, gsem1)

            return carry2

        lax.fori_loop(0, GK // 2, step, 0)
        pltpu.make_async_copy(rows0, acc.at[dstv.at[GK - 2]], ssem0).wait()
        pltpu.make_async_copy(rows1, acc.at[dstv.at[GK - 1]], ssem1).wait()
        return carry

    lax.fori_loop(0, kc // GK, group, 0)
    plsc.subcore_barrier()
    pltpu.sync_copy(acc.at[pl.ds(s * RPT, RPT)], out_hbm.at[c, pl.ds(s * RPT, RPT)])


def _sc_agg(hp, srcp, dstp):
    return pl.kernel(
        _agg_body,
        out_type=jax.ShapeDtypeStruct((NC, NPAD, H), jnp.float32),
        mesh=_mesh(),
        scratch_types=[
            pltpu.VMEM((GK, CH), jnp.int32),
            pltpu.VMEM((GK, CH), jnp.int32),
            pltpu.VMEM((CH, H), jnp.float32),
            pltpu.VMEM((CH, H), jnp.float32),
            pltpu.VMEM_SHARED((NPAD, H), jnp.float32),
            pltpu.SemaphoreType.DMA,
            pltpu.SemaphoreType.DMA,
            pltpu.SemaphoreType.DMA,
            pltpu.SemaphoreType.DMA,
        ],
    )(hp, srcp, dstp)


def _tc_a_body(x_ref, w_ref, d0_ref, d1_ref, dis_ref, h_ref):
    deg = d0_ref[...] + d1_ref[...] + 1.0
    dis = lax.rsqrt(deg)
    dis_ref[...] = dis
    h_ref[...] = dis * jnp.dot(x_ref[...], w_ref[...],
                               preferred_element_type=jnp.float32)


def _tc_a(x, w1, d0, d1):
    return pl.pallas_call(
        _tc_a_body,
        grid=(NB,),
        in_specs=[
            pl.BlockSpec((BR, D), lambda i: (i, 0)),
            pl.BlockSpec((D, H), lambda i: (0, 0)),
            pl.BlockSpec((BR, 1), lambda i: (i, 0)),
            pl.BlockSpec((BR, 1), lambda i: (i, 0)),
        ],
        out_specs=[
            pl.BlockSpec((BR, 1), lambda i: (i, 0)),
            pl.BlockSpec((BR, H), lambda i: (i, 0)),
        ],
        out_shape=[
            jax.ShapeDtypeStruct((N, 1), jnp.float32),
            jax.ShapeDtypeStruct((N, H), jnp.float32),
        ],
    )(x, w1, d0, d1)


def _tc_bc_body(a0_ref, a1_ref, hp_ref, dis_ref, b_ref, w_ref, out_ref):
    dis = dis_ref[...]
    h = jnp.maximum(dis * (a0_ref[...] + a1_ref[...] - hp_ref[...]) + b_ref[...], 0.0)
    out_ref[...] = dis * jnp.dot(h, w_ref[...], preferred_element_type=jnp.float32)


def _tc_bc(a0, a1, hp, dis, b, w):
    return pl.pallas_call(
        _tc_bc_body,
        grid=(NB,),
        in_specs=[
            pl.BlockSpec((BR, H), lambda i: (i, 0)),
            pl.BlockSpec((BR, H), lambda i: (i, 0)),
            pl.BlockSpec((BR, H), lambda i: (i, 0)),
            pl.BlockSpec((BR, 1), lambda i: (i, 0)),
            pl.BlockSpec((1, H), lambda i: (0, 0)),
            pl.BlockSpec((H, H), lambda i: (0, 0)),
        ],
        out_specs=pl.BlockSpec((BR, H), lambda i: (i, 0)),
        out_shape=jax.ShapeDtypeStruct((N, H), jnp.float32),
    )(a0, a1, hp, dis, b, w)


def _tc_d_body(a0_ref, a1_ref, hp_ref, dis_ref, b_ref, bt_ref,
               l1w_ref, l1b_ref, l2w_ref, l2b_ref, out_ref, sums, cnts):
    i = pl.program_id(0)

    @pl.when(i == 0)
    def _():
        sums[...] = jnp.zeros_like(sums)
        cnts[...] = jnp.zeros_like(cnts)

    dis = dis_ref[...]
    h = jnp.maximum(dis * (a0_ref[...] + a1_ref[...] - hp_ref[...]) + b_ref[...], 0.0)
    gi = lax.broadcasted_iota(jnp.int32, (BR, G), 1)
    S = (bt_ref[...] == gi).astype(jnp.float32)
    dn = (((0,), (0,)), ((), ()))
    sums[...] += lax.dot_general(S, h, dn, preferred_element_type=jnp.float32)
    cnts[...] += lax.dot_general(S, jnp.ones_like(h), dn,
                                 preferred_element_type=jnp.float32)

    @pl.when(i == NB - 1)
    def _():
        pooled = sums[...] / jnp.maximum(cnts[...], 1.0)
        z = jnp.maximum(jnp.dot(pooled, l1w_ref[...],
                                preferred_element_type=jnp.float32) + l1b_ref[...], 0.0)
        z = jnp.dot(z, l2w_ref[...], preferred_element_type=jnp.float32) + l2b_ref[...]
        m = jnp.max(z, axis=-1, keepdims=True)
        out_ref[...] = (z - m) - jnp.log(jnp.sum(jnp.exp(z - m), axis=-1, keepdims=True))


def _tc_d(a0, a1, hp, dis, b, bt, l1w, l1b, l2w, l2b):
    return pl.pallas_call(
        _tc_d_body,
        grid=(NB,),
        in_specs=[
            pl.BlockSpec((BR, H), lambda i: (i, 0)),
            pl.BlockSpec((BR, H), lambda i: (i, 0)),
            pl.BlockSpec((BR, H), lambda i: (i, 0)),
            pl.BlockSpec((BR, 1), lambda i: (i, 0)),
            pl.BlockSpec((1, H), lambda i: (0, 0)),
            pl.BlockSpec((BR, 1), lambda i: (i, 0)),
            pl.BlockSpec((H, H), lambda i: (0, 0)),
            pl.BlockSpec((1, H), lambda i: (0, 0)),
            pl.BlockSpec((H, C), lambda i: (0, 0)),
            pl.BlockSpec((1, C), lambda i: (0, 0)),
        ],
        out_specs=pl.BlockSpec((G, C), lambda i: (0, 0)),
        out_shape=jax.ShapeDtypeStruct((G, C), jnp.float32),
        scratch_shapes=[
            pltpu.VMEM((G, H), jnp.float32),
            pltpu.VMEM((G, H), jnp.float32),
        ],
    )(a0, a1, hp, dis, b, bt, l1w, l1b, l2w, l2b)


def kernel(x, edge_index, batch, target_size, W1, b1, W2, b2, W3, b3,
           lin1_w, lin1_b, lin2_w, lin2_b):
    del target_size
    pad = EPAD - E
    srcp = jnp.concatenate(
        [edge_index[0], jnp.zeros((pad,), edge_index.dtype)]).reshape(NW * K, CH)
    dstp = jnp.concatenate(
        [edge_index[1], jnp.full((pad,), N, edge_index.dtype)]).reshape(NW * K, CH)

    zeros_h = jnp.zeros((NPAD, H), jnp.float32)
    ones_h = jnp.ones((CH, H), jnp.float32)
    degp = _sc_deg(dstp, zeros_h, ones_h)
    d0 = degp[0, :N, 0:1]
    d1 = degp[1, :N, 0:1]

    dis, h1p = _tc_a(x, W1, d0, d1)
    a1 = _sc_agg(h1p, srcp, dstp)
    h2p = _tc_bc(a1[0, :N], a1[1, :N], h1p, dis, b1.reshape(1, H), W2)
    a2 = _sc_agg(h2p, srcp, dstp)
    h3p = _tc_bc(a2[0, :N], a2[1, :N], h2p, dis, b2.reshape(1, H), W3)
    a3 = _sc_agg(h3p, srcp, dstp)
    return _tc_d(a3[0, :N], a3[1, :N], h3p, dis, b3.reshape(1, H), batch.reshape(N, 1),
                 lin1_w, lin1_b.reshape(1, H), lin2_w, lin2_b.reshape(1, C))
